# baseline (speedup 1.0000x reference)
import functools

import jax
import jax.numpy as jnp
from jax import lax
from jax.experimental import pallas as pl
from jax.experimental.pallas import tpu as pltpu
from jax.experimental.pallas import tpu_sc as plsc

_GRID_W = 128
_NB = 8
_A = 4096
_CE = 256
_PD = 48
_PROWS = 4128
_TRASH = 1024


def _tc_body(emb_ref, w_ref, b_ref, out_ref):
    x = emb_ref[0, 0]
    res = jnp.dot(x, w_ref[...], preferred_element_type=jnp.float32)
    res = res + b_ref[...]
    out_ref[:1024, :] = res
    out_ref[1024:1032, :] = jnp.zeros((8, 64), jnp.float32)


def _tc_compute(emb, w64, bvec):
    return pl.pallas_call(
        _tc_body,
        grid=(_NB * 4,),
        in_specs=[
            pl.BlockSpec((1, 1, 1024, _CE), lambda i: (i // 4, i % 4, 0, 0)),
            pl.BlockSpec((_CE, 64), lambda i: (0, 0)),
            pl.BlockSpec((1, 64), lambda i: (0, 0)),
        ],
        out_specs=pl.BlockSpec((1032, 64), lambda i: (i, 0)),
        out_shape=jax.ShapeDtypeStruct((_NB * _PROWS, 64), jnp.float32),
    )(emb.reshape(_NB, 4, 1024, _CE), w64, bvec)


def _sc_body(p_hbm, idx_hbm, wts_hbm, offs_hbm, img_out, wimg_out,
             idx_v, wts_v, offs_v, tbl_v, wvals_v, cflat_v, cidx_v,
             posmap_v, gbuf_v, obuf_v, wimg_v, semA):
    c = lax.axis_index("c")
    s = lax.axis_index("s")
    wid = s * 2 + c
    b = wid // 4
    q = wid % 4

    pltpu.sync_copy(idx_hbm.at[b], idx_v)
    pltpu.sync_copy(wts_hbm.at[b], wts_v)
    pltpu.sync_copy(offs_hbm, offs_v)

    lanes = lax.iota(jnp.int32, 16)
    bb = jnp.full((16,), b, jnp.int32)
    hoff = plsc.load_gather(offs_v, [bb])
    woff = plsc.load_gather(offs_v, [bb + 8])
    row0 = hoff + q * 16

    trash16 = jnp.full((16,), _TRASH, jnp.int32)

    def _init(i, carry):
        base = i * 256
        for k in range(16):
            tbl_v[pl.ds(base + k * 16, 16)] = trash16
        return carry
    lax.fori_loop(0, 8, _init, 0)

    def _scat(i, carry):
        for k in range(4):
            off = i * 64 + k * 16
            iv = idx_v[pl.ds(off, 16)]
            gh = iv >> 7
            gw = iv & (_GRID_W - 1)
            lr = gh - row0
            valid = (lr >= 0) & (lr < 16)
            tix = jnp.where(valid, lr * _GRID_W + gw, 0)
            plsc.store_scatter(tbl_v, [tix], lanes + off, mask=valid)
        return carry
    lax.fori_loop(0, 64, _scat, 0)

    pbase = b * _PROWS

    def _patches(ii, off):
        for k in range(4):
            i = ii * 4 + k
            rl = i // 4
            ch = i % 4
            tix = rl * _GRID_W + woff + ch * 16 + lanes
            a = plsc.load_gather(tbl_v, [tix])
            wv = plsc.load_gather(wts_v, [jnp.minimum(a, _A - 1)])
            valid = a < _A
            wv = jnp.where(valid, wv, 0.0)
            wvals_v[pl.ds(rl * 64 + ch * 16, 16)] = wv
            rowid = pbase + a + ((a >> 10) << 3)
            plsc.store_compressed(cflat_v.at[pl.ds(off, 16)], rowid,
                                  mask=valid)
            vi = valid.astype(jnp.int32)
            pos = (off - 1) + plsc.cumsum(vi)
            pos = jnp.where(valid, pos, 1024)
            posmap_v[pl.ds(rl * 64 + ch * 16, 16)] = pos
            off = off + jnp.sum(vi)
        return off
    nval = lax.fori_loop(0, 16, _patches, jnp.int32(0))

    def _tail(i, carry):
        cflat_v[pl.ds(nval + i * 16, 16)] = jnp.full((16,), pbase + _TRASH,
                                                     jnp.int32)
        return carry
    lax.fori_loop(0, 8, _tail, 0)

    def _c2d(j, carry):
        for k in range(8):
            cidx_v[j, pl.ds(k * 16, 16)] = cflat_v[pl.ds(j * 128 + k * 16,
                                                         16)]
        return carry
    lax.fori_loop(0, 8, _c2d, 0)

    z16 = jnp.zeros((16,), jnp.float32)
    for k in range(4):
        gbuf_v[8, 0, pl.ds(k * 16, 16)] = z16

    nchunks = (nval + 127) >> 7

    def _fire(j, carry):
        pltpu.async_copy(p_hbm.at[cidx_v.at[j]], gbuf_v.at[j], semA)
        return carry
    lax.fori_loop(0, nchunks, _fire, 0)

    def _drain(j, carry):
        pltpu.make_async_copy(p_hbm.at[cidx_v.at[j]], gbuf_v.at[j],
                              semA).wait()
        return carry
    lax.fori_loop(0, nchunks, _drain, 0)

    lanes4 = lanes >> 2
    lanem3 = (lanes & 3) * 3

    def _quarter(q4, carry):
        def _xchunk(k, carry2):
            kpart = (k >> 3) * 1024 + (k & 7) * 16
            for rl_local in range(4):
                pvec = lanes4 + ((q4 * 4 + rl_local) * 64 + k * 4)
                pos = plsc.load_gather(posmap_v, [pvec])
                slotv = pos >> 7
                rowv = pos & 127
                wlv = plsc.load_gather(wvals_v, [pvec])
                for r in range(4):
                    lq = rl_local * 4 + r
                    lbase = (lq >> 3) * 2048 + (lq & 7) * 128
                    for ci in range(3):
                        x = plsc.load_gather(
                            gbuf_v, [slotv, rowv, lanem3 + (r * 12 + ci)])
                        obuf_v[pl.ds(kpart + lbase + ci * 4096, 16)] = x * wlv
            return carry2
        lax.fori_loop(0, 16, _xchunk, 0)
        for ci in range(3):
            pltpu.sync_copy(
                obuf_v.at[pl.ds(ci * 4096, 4096)],
                img_out.at[b, ci, pl.ds((q * 8 + q4 * 2) * 2048, 4096)])
        return carry
    lax.fori_loop(0, 4, _quarter, 0)

    def _wimg(ii, carry):
        for k in range(4):
            i = ii * 4 + k
            rl = i // 16
            ci = i % 16
            widx = rl * 64 + ((ci * 16 + lanes) >> 2)
            wvv = plsc.load_gather(wvals_v, [widx])
            for r in range(4):
                wimg_v[pl.ds((rl * 4 + r) * 256 + ci * 16, 16)] = wvv
        return carry
    lax.fori_loop(0, 64, _wimg, 0)
    pltpu.sync_copy(wimg_v, wimg_out.at[wid])


@functools.partial(
    pl.kernel,
    out_type=(
        jax.ShapeDtypeStruct((_NB, 3, 65536), jnp.float32),
        jax.ShapeDtypeStruct((32, 16384), jnp.float32),
    ),
    mesh=plsc.VectorSubcoreMesh(core_axis_name="c", subcore_axis_name="s"),
    compiler_params=pltpu.CompilerParams(needs_layout_passes=False,
                                         use_tc_tiling_on_sc=False),
    scratch_types=(
        pltpu.VMEM((_A,), jnp.int32),
        pltpu.VMEM((_A,), jnp.float32),
        pltpu.VMEM((128,), jnp.int32),
        pltpu.VMEM((16 * _GRID_W,), jnp.int32),
        pltpu.VMEM((1024,), jnp.float32),
        pltpu.VMEM((1168,), jnp.int32),
        pltpu.VMEM((8, 128), jnp.int32),
        pltpu.VMEM((1024,), jnp.int32),
        pltpu.VMEM((9, 128, 64), jnp.float32),
        pltpu.VMEM((12288 + 16,), jnp.float32),
        pltpu.VMEM((16384,), jnp.float32),
        pltpu.SemaphoreType.DMA,
    ),
)
def _sc_kernel(p_hbm, idx_hbm, wts_hbm, offs_hbm, img_out, wimg_out,
               idx_v, wts_v, offs_v, tbl_v, wvals_v, cflat_v, cidx_v,
               posmap_v, gbuf_v, obuf_v, wimg_v, semA):
    _sc_body(p_hbm, idx_hbm, wts_hbm, offs_hbm, img_out, wimg_out,
             idx_v, wts_v, offs_v, tbl_v, wvals_v, cflat_v, cidx_v,
             posmap_v, gbuf_v, obuf_v, wimg_v, semA)


def kernel(emb, weights, W, b, indexes, h_offset, w_offset, img_h, img_w):
    del img_h, img_w
    w64 = jnp.concatenate([W, jnp.zeros((_CE, 64 - _PD), W.dtype)], axis=1)
    b64 = jnp.concatenate([b, jnp.zeros((64 - _PD,), b.dtype)]).reshape(1, 64)
    p = _tc_compute(emb, w64, b64)
    offs = jnp.concatenate([h_offset.astype(jnp.int32),
                            w_offset.astype(jnp.int32),
                            jnp.zeros((112,), jnp.int32)])
    img4, wimg4 = _sc_kernel(p, indexes, weights, offs)
    img = (img4.reshape(_NB, 3, 32, 2, 8, 128)
           .transpose(0, 2, 4, 3, 5, 1)
           .reshape(_NB, 256, 256, 3))
    wimg = wimg4.reshape(_NB, 256, 256, 1)
    return img, wimg


# speedup vs baseline: 1.5563x; 1.5563x over previous
import functools

import jax
import jax.numpy as jnp
from jax import lax
from jax.experimental import pallas as pl
from jax.experimental.pallas import tpu as pltpu
from jax.experimental.pallas import tpu_sc as plsc

_GRID_W = 128
_NB = 8
_A = 4096
_CE = 256
_PD = 48
_PROWS = 4128
_ZROW = 1024


def _tc_body(emb_ref, w_ref, b_ref, out_ref):
    x = emb_ref[0, 0]
    res = jnp.dot(x, w_ref[...], preferred_element_type=jnp.float32)
    res = res + b_ref[...]
    out_ref[:1024, :] = res
    out_ref[1024:1032, :] = jnp.zeros((8, 64), jnp.float32)


def _tc_compute(emb, w64, bvec):
    return pl.pallas_call(
        _tc_body,
        grid=(_NB * 4,),
        in_specs=[
            pl.BlockSpec((1, 1, 1024, _CE), lambda i: (i // 4, i % 4, 0, 0)),
            pl.BlockSpec((_CE, 64), lambda i: (0, 0)),
            pl.BlockSpec((1, 64), lambda i: (0, 0)),
        ],
        out_specs=pl.BlockSpec((1032, 64), lambda i: (i, 0)),
        out_shape=jax.ShapeDtypeStruct((_NB * _PROWS, 64), jnp.float32),
    )(emb.reshape(_NB, 4, 1024, _CE), w64, bvec)


def _sc_body(p_hbm, idx_hbm, wts_hbm, offs_hbm, img_out, wimg_out,
             idx_v, wts_v, offs_v, tbl_v, wvals_v, cflat_v, cidx_v,
             posmap_v, gbuf_v, obuf_v, wimg_v, semA):
    c = lax.axis_index("c")
    s = lax.axis_index("s")
    wid = s * 2 + c
    b = wid // 4
    q = wid % 4

    pltpu.sync_copy(idx_hbm.at[b], idx_v)
    pltpu.sync_copy(wts_hbm.at[b], wts_v)
    pltpu.sync_copy(offs_hbm, offs_v)

    lanes = lax.iota(jnp.int32, 16)
    bb = jnp.full((16,), b, jnp.int32)
    hoff = plsc.load_gather(offs_v, [bb])
    woff = plsc.load_gather(offs_v, [bb + 8])
    row0 = hoff + q * 16

    trash16 = jnp.full((16,), _A, jnp.int32)

    def _init(i, carry):
        base = i * 256
        for k in range(16):
            tbl_v[pl.ds(base + k * 16, 16)] = trash16
        return carry
    lax.fori_loop(0, 8, _init, 0)

    def _scat(i, carry):
        for k in range(4):
            off = i * 64 + k * 16
            iv = idx_v[pl.ds(off, 16)]
            gh = iv >> 7
            gw = iv & (_GRID_W - 1)
            lr = gh - row0
            valid = (lr >= 0) & (lr < 16)
            tix = jnp.where(valid, lr * _GRID_W + gw, 0)
            plsc.store_scatter(tbl_v, [tix], lanes + off, mask=valid)
        return carry
    lax.fori_loop(0, 64, _scat, 0)

    pbase = b * _PROWS

    def _patches(ii, off):
        for k in range(4):
            i = ii * 4 + k
            rl = i // 4
            ch = i % 4
            tix = rl * _GRID_W + woff + ch * 16 + lanes
            a = plsc.load_gather(tbl_v, [tix])
            wv = plsc.load_gather(wts_v, [jnp.minimum(a, _A - 1)])
            valid = a < _A
            wv = jnp.where(valid, wv, 0.0)
            wvals_v[pl.ds(rl * 64 + ch * 16, 16)] = wv
            rowid = pbase + a + ((a >> 10) << 3)
            plsc.store_compressed(cflat_v.at[pl.ds(off, 16)], rowid,
                                  mask=valid)
            vi = valid.astype(jnp.int32)
            pos = (off - 1) + plsc.cumsum(vi)
            pos = jnp.where(valid, pos, 1024)
            posmap_v[pl.ds(rl * 64 + ch * 16, 16)] = pos
            off = off + jnp.sum(vi)
        return off
    nval = lax.fori_loop(0, 16, _patches, jnp.int32(0))

    def _tail(i, carry):
        cflat_v[pl.ds(nval + i * 16, 16)] = jnp.full((16,), pbase + _ZROW,
                                                     jnp.int32)
        return carry
    lax.fori_loop(0, 8, _tail, 0)

    def _c2d(j, carry):
        for k in range(8):
            cidx_v[j, pl.ds(k * 16, 16)] = cflat_v[pl.ds(j * 128 + k * 16,
                                                         16)]
        return carry
    lax.fori_loop(0, 8, _c2d, 0)

    z16 = jnp.zeros((16,), jnp.float32)
    for k in range(4):
        gbuf_v[8, 0, pl.ds(k * 16, 16)] = z16

    nchunks = (nval + 127) >> 7

    def _fire(j, carry):
        pltpu.async_copy(p_hbm.at[cidx_v.at[j]], gbuf_v.at[j], semA)
        return carry
    lax.fori_loop(0, nchunks, _fire, 0)

    def _drain(j, carry):
        pltpu.make_async_copy(p_hbm.at[cidx_v.at[j]], gbuf_v.at[j],
                              semA).wait()
        return carry
    lax.fori_loop(0, nchunks, _drain, 0)

    lanes4 = lanes >> 2
    lanem3 = (lanes & 3) * 3

    def _quarter(q4, carry):
        def _xchunk(k, carry2):
            kpart = (k >> 3) * 1024 + (k & 7) * 16
            for rl_local in range(4):
                pvec = lanes4 + ((q4 * 4 + rl_local) * 64 + k * 4)
                pos = plsc.load_gather(posmap_v, [pvec])
                slotv = pos >> 7
                rowv = pos & 127
                wlv = plsc.load_gather(wvals_v, [pvec])
                for r in range(4):
                    lq = rl_local * 4 + r
                    lbase = (lq >> 3) * 2048 + (lq & 7) * 128
                    for ci in range(3):
                        x = plsc.load_gather(
                            gbuf_v, [slotv, rowv, lanem3 + (r * 12 + ci)])
                        obuf_v[pl.ds(kpart + lbase + ci * 4096, 16)] = x * wlv
            return carry2
        lax.fori_loop(0, 16, _xchunk, 0)
        for ci in range(3):
            pltpu.sync_copy(
                obuf_v.at[pl.ds(ci * 4096, 4096)],
                img_out.at[b, ci, pl.ds((q * 8 + q4 * 2) * 2048, 4096)])
        return carry
    lax.fori_loop(0, 4, _quarter, 0)

    def _wimg(ii, carry):
        for k in range(4):
            i = ii * 4 + k
            rl = i // 16
            ci = i % 16
            widx = rl * 64 + ((ci * 16 + lanes) >> 2)
            wvv = plsc.load_gather(wvals_v, [widx])
            for r in range(4):
                wimg_v[pl.ds((rl * 4 + r) * 256 + ci * 16, 16)] = wvv
        return carry
    lax.fori_loop(0, 64, _wimg, 0)
    pltpu.sync_copy(wimg_v, wimg_out.at[wid])


@functools.partial(
    pl.kernel,
    out_type=(
        jax.ShapeDtypeStruct((_NB, 3, 65536), jnp.float32),
        jax.ShapeDtypeStruct((32, 16384), jnp.float32),
    ),
    mesh=plsc.VectorSubcoreMesh(core_axis_name="c", subcore_axis_name="s"),
    compiler_params=pltpu.CompilerParams(needs_layout_passes=False,
                                         use_tc_tiling_on_sc=False),
    scratch_types=(
        pltpu.VMEM((_A,), jnp.int32),
        pltpu.VMEM((_A,), jnp.float32),
        pltpu.VMEM((128,), jnp.int32),
        pltpu.VMEM((16 * _GRID_W,), jnp.int32),
        pltpu.VMEM((1024,), jnp.float32),
        pltpu.VMEM((1168,), jnp.int32),
        pltpu.VMEM((8, 128), jnp.int32),
        pltpu.VMEM((1024,), jnp.int32),
        pltpu.VMEM((9, 128, 64), jnp.float32),
        pltpu.VMEM((12288 + 16,), jnp.float32),
        pltpu.VMEM((16384,), jnp.float32),
        pltpu.SemaphoreType.DMA,
    ),
)
def _sc_kernel(p_hbm, idx_hbm, wts_hbm, offs_hbm, img_out, wimg_out,
               idx_v, wts_v, offs_v, tbl_v, wvals_v, cflat_v, cidx_v,
               posmap_v, gbuf_v, obuf_v, wimg_v, semA):
    _sc_body(p_hbm, idx_hbm, wts_hbm, offs_hbm, img_out, wimg_out,
             idx_v, wts_v, offs_v, tbl_v, wvals_v, cflat_v, cidx_v,
             posmap_v, gbuf_v, obuf_v, wimg_v, semA)


def kernel(emb, weights, W, b, indexes, h_offset, w_offset, img_h, img_w):
    del img_h, img_w
    w64 = jnp.concatenate([W, jnp.zeros((_CE, 64 - _PD), W.dtype)], axis=1)
    b64 = jnp.concatenate([b, jnp.zeros((64 - _PD,), b.dtype)]).reshape(1, 64)
    p = _tc_compute(emb, w64, b64)
    offs = jnp.concatenate([h_offset.astype(jnp.int32),
                            w_offset.astype(jnp.int32),
                            jnp.zeros((112,), jnp.int32)])
    img4, wimg4 = _sc_kernel(p, indexes, weights, offs)
    img = (img4.reshape(_NB, 3, 32, 2, 8, 128)
           .transpose(0, 2, 4, 3, 5, 1)
           .reshape(_NB, 256, 256, 3))
    wimg = wimg4.reshape(_NB, 256, 256, 1)
    return img, wimg


# speedup vs baseline: 1.7667x; 1.1352x over previous
import functools

import jax
import jax.numpy as jnp
from jax import lax
from jax.experimental import pallas as pl
from jax.experimental.pallas import tpu as pltpu
from jax.experimental.pallas import tpu_sc as plsc

_GRID_W = 128
_NB = 8
_A = 4096
_CE = 256
_PD = 48
_PROWS = 4104
_ZROW = _A


def _tc_body(emb_ref, w_ref, b_ref, out_ref):
    x = emb_ref[0]
    res = jnp.dot(x, w_ref[...], preferred_element_type=jnp.float32)
    res = res + b_ref[...]
    out_ref[:_A, :] = res
    out_ref[_A:_PROWS, :] = jnp.zeros((_PROWS - _A, 64), jnp.float32)


def _tc_compute(emb, w64, bvec):
    return pl.pallas_call(
        _tc_body,
        grid=(_NB,),
        in_specs=[
            pl.BlockSpec((1, _A, _CE), lambda i: (i, 0, 0)),
            pl.BlockSpec((_CE, 64), lambda i: (0, 0)),
            pl.BlockSpec((1, 64), lambda i: (0, 0)),
        ],
        out_specs=pl.BlockSpec((_PROWS, 64), lambda i: (i, 0)),
        out_shape=jax.ShapeDtypeStruct((_NB * _PROWS, 64), jnp.float32),
    )(emb, w64, bvec)


def _sc_body(p_hbm, idx_hbm, wts_hbm, offs_hbm, img_out, wimg_out,
             idx_v, wts_v, offs_v, tbl_v, wvals_v, cflat_v, cidx_v,
             posmap_v, gbuf_v, obuf_v, wimg_v, semA):
    c = lax.axis_index("c")
    s = lax.axis_index("s")
    wid = s * 2 + c
    b = wid // 4
    q = wid % 4

    pltpu.sync_copy(idx_hbm.at[b], idx_v)
    pltpu.sync_copy(wts_hbm.at[b], wts_v)
    pltpu.sync_copy(offs_hbm, offs_v)

    lanes = lax.iota(jnp.int32, 16)
    bb = jnp.full((16,), b, jnp.int32)
    hoff = plsc.load_gather(offs_v, [bb])
    woff = plsc.load_gather(offs_v, [bb + 8])
    row0 = hoff + q * 16

    trash16 = jnp.full((16,), _A, jnp.int32)

    def _init(i, carry):
        base = i * 256
        for k in range(16):
            tbl_v[pl.ds(base + k * 16, 16)] = trash16
        return carry
    lax.fori_loop(0, 8, _init, 0)

    def _scat(i, carry):
        for k in range(4):
            off = i * 64 + k * 16
            iv = idx_v[pl.ds(off, 16)]
            gh = iv >> 7
            gw = iv & (_GRID_W - 1)
            lr = gh - row0
            valid = (lr >= 0) & (lr < 16)
            tix = jnp.where(valid, lr * _GRID_W + gw, 0)
            plsc.store_scatter(tbl_v, [tix], lanes + off, mask=valid)
        return carry
    lax.fori_loop(0, 64, _scat, 0)

    pbase = b * _PROWS

    def _patches(ii, off):
        for k in range(4):
            i = ii * 4 + k
            rl = i // 4
            ch = i % 4
            tix = rl * _GRID_W + woff + ch * 16 + lanes
            a = plsc.load_gather(tbl_v, [tix])
            wv = plsc.load_gather(wts_v, [jnp.minimum(a, _A - 1)])
            valid = a < _A
            wv = jnp.where(valid, wv, 0.0)
            wvals_v[pl.ds(rl * 64 + ch * 16, 16)] = wv
            plsc.store_compressed(cflat_v.at[pl.ds(off, 16)], pbase + a,
                                  mask=valid)
            vi = valid.astype(jnp.int32)
            pos = (off - 1) + plsc.cumsum(vi)
            pos = jnp.where(valid, pos, 1024)
            posmap_v[pl.ds(rl * 64 + ch * 16, 16)] = pos
            off = off + jnp.sum(vi)
        return off
    nval = lax.fori_loop(0, 16, _patches, jnp.int32(0))

    def _tail(i, carry):
        cflat_v[pl.ds(nval + i * 16, 16)] = jnp.full((16,), pbase + _ZROW,
                                                     jnp.int32)
        return carry
    lax.fori_loop(0, 8, _tail, 0)

    def _c2d(j, carry):
        for k in range(8):
            cidx_v[j, pl.ds(k * 16, 16)] = cflat_v[pl.ds(j * 128 + k * 16,
                                                         16)]
        return carry
    lax.fori_loop(0, 8, _c2d, 0)

    z16 = jnp.zeros((16,), jnp.float32)
    for k in range(4):
        gbuf_v[8, 0, pl.ds(k * 16, 16)] = z16

    nchunks = (nval + 127) >> 7

    def _fire(j, carry):
        pltpu.async_copy(p_hbm.at[cidx_v.at[j]], gbuf_v.at[j], semA)
        return carry
    lax.fori_loop(0, nchunks, _fire, 0)

    def _drain(j, carry):
        pltpu.make_async_copy(p_hbm.at[cidx_v.at[j]], gbuf_v.at[j],
                              semA).wait()
        return carry
    lax.fori_loop(0, nchunks, _drain, 0)

    lanes4 = lanes >> 2
    lanem3 = (lanes & 3) * 3

    def _quarter(q4, carry):
        def _xchunk(k, carry2):
            kpart = (k >> 3) * 1024 + (k & 7) * 16
            for rl_local in range(4):
                pvec = lanes4 + ((q4 * 4 + rl_local) * 64 + k * 4)
                pos = plsc.load_gather(posmap_v, [pvec])
                slotv = pos >> 7
                rowv = pos & 127
                wlv = plsc.load_gather(wvals_v, [pvec])
                for r in range(4):
                    lq = rl_local * 4 + r
                    lbase = (lq >> 3) * 2048 + (lq & 7) * 128
                    for ci in range(3):
                        x = plsc.load_gather(
                            gbuf_v, [slotv, rowv, lanem3 + (r * 12 + ci)])
                        obuf_v[pl.ds(kpart + lbase + ci * 4096, 16)] = x * wlv
            return carry2
        lax.fori_loop(0, 16, _xchunk, 0)
        for ci in range(3):
            pltpu.sync_copy(
                obuf_v.at[pl.ds(ci * 4096, 4096)],
                img_out.at[b, ci, pl.ds((q * 8 + q4 * 2) * 2048, 4096)])
        return carry
    lax.fori_loop(0, 4, _quarter, 0)

    def _wimg(ii, carry):
        for k in range(4):
            i = ii * 4 + k
            rl = i // 16
            ci = i % 16
            widx = rl * 64 + ((ci * 16 + lanes) >> 2)
            wvv = plsc.load_gather(wvals_v, [widx])
            for r in range(4):
                wimg_v[pl.ds((rl * 4 + r) * 256 + ci * 16, 16)] = wvv
        return carry
    lax.fori_loop(0, 64, _wimg, 0)
    pltpu.sync_copy(wimg_v, wimg_out.at[wid])


@functools.partial(
    pl.kernel,
    out_type=(
        jax.ShapeDtypeStruct((_NB, 3, 65536), jnp.float32),
        jax.ShapeDtypeStruct((32, 16384), jnp.float32),
    ),
    mesh=plsc.VectorSubcoreMesh(core_axis_name="c", subcore_axis_name="s"),
    compiler_params=pltpu.CompilerParams(needs_layout_passes=False,
                                         use_tc_tiling_on_sc=False),
    scratch_types=(
        pltpu.VMEM((_A,), jnp.int32),
        pltpu.VMEM((_A,), jnp.float32),
        pltpu.VMEM((128,), jnp.int32),
        pltpu.VMEM((16 * _GRID_W,), jnp.int32),
        pltpu.VMEM((1024,), jnp.float32),
        pltpu.VMEM((1168,), jnp.int32),
        pltpu.VMEM((8, 128), jnp.int32),
        pltpu.VMEM((1024,), jnp.int32),
        pltpu.VMEM((9, 128, 64), jnp.float32),
        pltpu.VMEM((12288 + 16,), jnp.float32),
        pltpu.VMEM((16384,), jnp.float32),
        pltpu.SemaphoreType.DMA,
    ),
)
def _sc_kernel(p_hbm, idx_hbm, wts_hbm, offs_hbm, img_out, wimg_out,
               idx_v, wts_v, offs_v, tbl_v, wvals_v, cflat_v, cidx_v,
               posmap_v, gbuf_v, obuf_v, wimg_v, semA):
    _sc_body(p_hbm, idx_hbm, wts_hbm, offs_hbm, img_out, wimg_out,
             idx_v, wts_v, offs_v, tbl_v, wvals_v, cflat_v, cidx_v,
             posmap_v, gbuf_v, obuf_v, wimg_v, semA)


def kernel(emb, weights, W, b, indexes, h_offset, w_offset, img_h, img_w):
    del img_h, img_w
    w64 = jnp.concatenate([W, jnp.zeros((_CE, 64 - _PD), W.dtype)], axis=1)
    b64 = jnp.concatenate([b, jnp.zeros((64 - _PD,), b.dtype)]).reshape(1, 64)
    p = _tc_compute(emb, w64, b64)
    offs = jnp.concatenate([h_offset.astype(jnp.int32),
                            w_offset.astype(jnp.int32),
                            jnp.zeros((112,), jnp.int32)])
    img4, wimg4 = _sc_kernel(p, indexes, weights, offs)
    img = (img4.reshape(_NB, 3, 32, 2, 8, 128)
           .transpose(0, 2, 4, 3, 5, 1)
           .reshape(_NB, 256, 256, 3))
    wimg = wimg4.reshape(_NB, 256, 256, 1)
    return img, wimg
